# baseline (speedup 1.0000x reference)
import functools

import jax
import jax.numpy as jnp
from jax import lax
from jax.experimental import pallas as pl
from jax.experimental.pallas import tpu as pltpu
from jax.experimental.pallas import tpu_sc as plsc

_V = 1000000
_D = 64
_B = 4096
_S = 200

_PBLK = 8192
_NSTEP = pl.cdiv(_V // 2, _PBLK)
_CHUNK = 2 * _PBLK


def _rowdot_body(tbl_ref, w_ref, b_ref, out_ref):
    x = tbl_ref[...]
    acc = lax.dot_general(w_ref[...], x, (((1,), (1,)), ((), ())),
                          preferred_element_type=jnp.float32)
    out_ref[...] = ((acc + b_ref[0]) * (1.0 / _S))[None]


def _rowdot(table, W, b):
    tp = table.reshape(_V // 2, 2 * _D)
    w8 = jnp.zeros((8, 2 * _D), jnp.float32)
    w8 = w8.at[0, :_D].set(W.reshape(-1)).at[1, _D:].set(W.reshape(-1))
    out = pl.pallas_call(
        _rowdot_body,
        grid=(_NSTEP,),
        in_specs=[
            pl.BlockSpec((_PBLK, 2 * _D), lambda i: (i, 0)),
            pl.BlockSpec((8, 2 * _D), lambda i: (0, 0)),
            pl.BlockSpec(memory_space=pltpu.SMEM),
        ],
        out_specs=pl.BlockSpec((1, 8, _PBLK), lambda i: (i, 0, 0)),
        out_shape=jax.ShapeDtypeStruct((_NSTEP, 8, _PBLK), jnp.float32),
    )(tp, w8, b)
    return out.reshape(-1)


def _make_gather_kernel():
    info = plsc.get_sparse_core_info()
    nc, ns = info.num_cores, info.num_subcores
    nw = nc * ns
    rows_per_w = _B // nw
    idx_per_w = rows_per_w * _S
    n_grp = rows_per_w // 16
    nvr = 2 * _S // 16

    mesh = plsc.VectorSubcoreMesh(core_axis_name="c", subcore_axis_name="s")

    @functools.partial(
        pl.kernel,
        out_type=jax.ShapeDtypeStruct((_B,), jnp.float32),
        mesh=mesh,
        scratch_types=[
            pltpu.VMEM((idx_per_w,), jnp.int32),
            pltpu.VMEM((idx_per_w,), jnp.float32),
            pltpu.VMEM((rows_per_w,), jnp.float32),
            pltpu.SemaphoreType.DMA,
        ],
    )
    def gather_reduce(xt_hbm, t_hbm, out_hbm, idx_v, vals_v, out_v, sem):
        wid = lax.axis_index("s") * nc + lax.axis_index("c")
        base = wid * idx_per_w
        pltpu.sync_copy(xt_hbm.at[pl.ds(base, idx_per_w)], idx_v)
        pltpu.async_copy(t_hbm.at[idx_v], vals_v, sem).wait()

        def body(c, accs):
            off = c * rows_per_w
            return tuple(
                accs[g] + vals_v[pl.ds(off + g * 16, 16)]
                for g in range(n_grp)
            )

        accs = lax.fori_loop(
            0, _S, body,
            tuple(jnp.zeros((16,), jnp.float32) for _ in range(n_grp)))
        for g in range(n_grp):
            y = 1.0 / (1.0 + jnp.exp(-accs[g]))
            out_v[pl.ds(g * 16, 16)] = y

        pltpu.sync_copy(out_v,
                        out_hbm.at[pl.ds(wid * rows_per_w, rows_per_w)])

    return gather_reduce


def _transpose_body(x_ref, out_ref):
    v = x_ref[...]
    u = jnp.bitwise_and(v, _CHUNK - 1)
    addr = jnp.left_shift(v - u, 2) + \
        jnp.left_shift(jnp.bitwise_and(u, 1), 13) + jnp.right_shift(u, 1)
    out_ref[...] = addr.T[None]


def _transpose_idx(x):
    nw = 32
    rpw = _B // nw
    return pl.pallas_call(
        _transpose_body,
        grid=(nw,),
        in_specs=[pl.BlockSpec((rpw, _S), lambda i: (i, 0))],
        out_specs=pl.BlockSpec((1, _S, rpw), lambda i: (i, 0, 0)),
        out_shape=jax.ShapeDtypeStruct((nw, _S, rpw), jnp.int32),
    )(x)


def kernel(x, table, W, b):
    t = _rowdot(table, W, b)
    xt = _transpose_idx(x)
    gk = _make_gather_kernel()
    out = gk(xt.reshape(-1), t)
    return out.reshape(_B, 1)


# speedup vs baseline: 4.9667x; 4.9667x over previous
import functools

import jax
import jax.numpy as jnp
from jax import lax
from jax.experimental import pallas as pl
from jax.experimental.pallas import tpu as pltpu
from jax.experimental.pallas import tpu_sc as plsc

_V = 1000000
_D = 64
_B = 4096
_S = 200

_CBLK = 16384
_NSTEP = pl.cdiv(_V, _CBLK)


def _rowdot_body(tbl_ref, w_ref, b_ref, out_ref):
    x = tbl_ref[...]
    acc = jnp.dot(w_ref[...], x,
                  preferred_element_type=jnp.float32)
    out_ref[...] = ((acc[0:1] + b_ref[0]) * (1.0 / _S))[None]


def _rowdot(table, W, b):
    tt = table.T
    w8 = jnp.broadcast_to(W.reshape(1, _D), (8, _D))
    out = pl.pallas_call(
        _rowdot_body,
        grid=(_NSTEP,),
        in_specs=[
            pl.BlockSpec((_D, _CBLK), lambda i: (0, i)),
            pl.BlockSpec((8, _D), lambda i: (0, 0)),
            pl.BlockSpec(memory_space=pltpu.SMEM),
        ],
        out_specs=pl.BlockSpec((1, 1, _CBLK), lambda i: (i, 0, 0)),
        out_shape=jax.ShapeDtypeStruct((_NSTEP, 1, _CBLK), jnp.float32),
    )(tt, w8, b)
    return out.reshape(-1)


def _make_gather_kernel():
    info = plsc.get_sparse_core_info()
    nc, ns = info.num_cores, info.num_subcores
    nw = nc * ns
    rows_per_w = _B // nw
    idx_per_w = rows_per_w * _S
    n_grp = rows_per_w // 16

    mesh = plsc.VectorSubcoreMesh(core_axis_name="c", subcore_axis_name="s")

    @functools.partial(
        pl.kernel,
        out_type=jax.ShapeDtypeStruct((_B,), jnp.float32),
        mesh=mesh,
        scratch_types=[
            pltpu.VMEM((idx_per_w,), jnp.int32),
            pltpu.VMEM((idx_per_w,), jnp.float32),
            pltpu.VMEM((rows_per_w,), jnp.float32),
            pltpu.SemaphoreType.DMA,
        ],
    )
    def gather_reduce(xt_hbm, t_hbm, out_hbm, idx_v, vals_v, out_v, sem):
        wid = lax.axis_index("s") * nc + lax.axis_index("c")
        base = wid * idx_per_w
        pltpu.sync_copy(xt_hbm.at[pl.ds(base, idx_per_w)], idx_v)
        pltpu.async_copy(t_hbm.at[idx_v], vals_v, sem).wait()

        def body(c, accs):
            off = c * rows_per_w
            return tuple(
                accs[g] + vals_v[pl.ds(off + g * 16, 16)]
                for g in range(n_grp)
            )

        accs = lax.fori_loop(
            0, _S, body,
            tuple(jnp.zeros((16,), jnp.float32) for _ in range(n_grp)))
        for g in range(n_grp):
            y = 1.0 / (1.0 + jnp.exp(-accs[g]))
            out_v[pl.ds(g * 16, 16)] = y

        pltpu.sync_copy(out_v,
                        out_hbm.at[pl.ds(wid * rows_per_w, rows_per_w)])

    return gather_reduce


def _shuffle_body(xv_ref, out_ref):
    out_ref[...] = xv_ref[...][None]


def _shuffle_idx(x):
    nw = 32
    rpw = _B // nw
    return pl.pallas_call(
        _shuffle_body,
        grid=(nw,),
        in_specs=[pl.BlockSpec((_S, rpw), lambda w: (0, w))],
        out_specs=pl.BlockSpec((1, _S, rpw), lambda w: (w, 0, 0)),
        out_shape=jax.ShapeDtypeStruct((nw, _S, rpw), jnp.int32),
    )(x.T)


def kernel(x, table, W, b):
    t = _rowdot(table, W, b)
    xt = _shuffle_idx(x)
    gk = _make_gather_kernel()
    out = gk(xt.reshape(-1), t)
    return out.reshape(_B, 1)


# speedup vs baseline: 5.3308x; 1.0733x over previous
import functools

import jax
import jax.numpy as jnp
from jax import lax
from jax.experimental import pallas as pl
from jax.experimental.pallas import tpu as pltpu
from jax.experimental.pallas import tpu_sc as plsc

_V = 1000000
_D = 64
_B = 4096
_S = 200

_CBLK = 32768
_NSTEP = pl.cdiv(_V, _CBLK)


def _rowdot_body(tbl_ref, w_ref, b_ref, out_ref):
    x = tbl_ref[...]
    acc = jnp.dot(w_ref[...], x,
                  preferred_element_type=jnp.float32)
    out_ref[...] = ((acc[0:1] + b_ref[0]) * (1.0 / _S))[None]


def _rowdot(table, W, b):
    tt = table.T
    w8 = jnp.broadcast_to(W.reshape(1, _D), (8, _D))
    out = pl.pallas_call(
        _rowdot_body,
        grid=(_NSTEP,),
        in_specs=[
            pl.BlockSpec((_D, _CBLK), lambda i: (0, i)),
            pl.BlockSpec((8, _D), lambda i: (0, 0)),
            pl.BlockSpec(memory_space=pltpu.SMEM),
        ],
        out_specs=pl.BlockSpec((1, 1, _CBLK), lambda i: (i, 0, 0)),
        out_shape=jax.ShapeDtypeStruct((_NSTEP, 1, _CBLK), jnp.float32),
    )(tt, w8, b)
    return out.reshape(-1)


def _make_gather_kernel():
    info = plsc.get_sparse_core_info()
    nc, ns = info.num_cores, info.num_subcores
    nw = nc * ns
    rows_per_w = _B // nw
    idx_per_w = rows_per_w * _S
    n_grp = rows_per_w // 16

    mesh = plsc.VectorSubcoreMesh(core_axis_name="c", subcore_axis_name="s")

    @functools.partial(
        pl.kernel,
        out_type=jax.ShapeDtypeStruct((_B,), jnp.float32),
        mesh=mesh,
        scratch_types=[
            pltpu.VMEM((idx_per_w,), jnp.int32),
            pltpu.VMEM((idx_per_w,), jnp.float32),
            pltpu.VMEM((rows_per_w,), jnp.float32),
            pltpu.SemaphoreType.DMA,
        ],
    )
    def gather_reduce(xt_hbm, t_hbm, out_hbm, idx_v, vals_v, out_v, sem):
        wid = lax.axis_index("s") * nc + lax.axis_index("c")
        base = wid * idx_per_w
        pltpu.sync_copy(xt_hbm.at[pl.ds(base, idx_per_w)], idx_v)
        pltpu.async_copy(t_hbm.at[idx_v], vals_v, sem).wait()

        def body(c, accs):
            off = c * rows_per_w
            return tuple(
                accs[g] + vals_v[pl.ds(off + g * 16, 16)]
                for g in range(n_grp)
            )

        accs = lax.fori_loop(
            0, _S, body,
            tuple(jnp.zeros((16,), jnp.float32) for _ in range(n_grp)))
        for g in range(n_grp):
            y = 1.0 / (1.0 + jnp.exp(-accs[g]))
            out_v[pl.ds(g * 16, 16)] = y

        pltpu.sync_copy(out_v,
                        out_hbm.at[pl.ds(wid * rows_per_w, rows_per_w)])

    return gather_reduce


def _shuffle_body(xv_ref, out_ref):
    out_ref[...] = xv_ref[...][None]


def _shuffle_idx(x):
    nw = 32
    rpw = _B // nw
    return pl.pallas_call(
        _shuffle_body,
        grid=(nw,),
        in_specs=[pl.BlockSpec((_S, rpw), lambda w: (0, w))],
        out_specs=pl.BlockSpec((1, _S, rpw), lambda w: (w, 0, 0)),
        out_shape=jax.ShapeDtypeStruct((nw, _S, rpw), jnp.int32),
    )(x.T)


def kernel(x, table, W, b):
    t = _rowdot(table, W, b)
    xt = _shuffle_idx(x)
    gk = _make_gather_kernel()
    out = gk(xt.reshape(-1), t)
    return out.reshape(_B, 1)
